# Initial kernel scaffold; baseline (speedup 1.0000x reference)
import jax
import jax.numpy as jnp
from jax.experimental import pallas as pl


def kernel(encoder_out, align_phone, pitch, beats, W_pitch, b_pitch, W_pos, b_pos, W_beats, b_beats):
    raise NotImplementedError("write your pallas kernel here")


# speedup vs baseline: 3.3150x; 3.3150x over previous
import functools

import jax
import jax.numpy as jnp
from jax import lax
from jax.experimental import pallas as pl
from jax.experimental.pallas import tpu as pltpu
from jax.experimental.pallas import tpu_sc as plsc

_B, _P, _F, _H = 16, 512, 2048, 1024


def _idx_kernel(ap_ref, out_ref):
    x = ap_ref[...]
    prev = jnp.concatenate([x[:, :1], x[:, :-1]], axis=1)
    c = (x != prev).astype(jnp.int32)
    k = 1
    while k < _F:
        shifted = jnp.concatenate(
            [jnp.zeros((_B, k), jnp.int32), c[:, : _F - k]], axis=1)
        c = c + shifted
        k *= 2
    row = lax.broadcasted_iota(jnp.int32, (_B, _F), 0)
    out_ref[...] = c + row * _P


def _build_indices(ap):
    return pl.pallas_call(
        _idx_kernel,
        out_shape=jax.ShapeDtypeStruct((_B, _F), jnp.int32),
    )(ap)


_NW = 32
_ROWS_PER_W = (_B * _F) // _NW
_CHUNK = 64
_N_IT = _ROWS_PER_W // _CHUNK


def _sc_gather(table, gidx):
    mesh = plsc.VectorSubcoreMesh(core_axis_name="c", subcore_axis_name="s")

    @functools.partial(
        pl.kernel,
        mesh=mesh,
        out_type=jax.ShapeDtypeStruct((_B * _F, _H), jnp.float32),
        scratch_types=[
            pltpu.VMEM((_ROWS_PER_W,), jnp.int32),
            pltpu.VMEM((_CHUNK, _H), jnp.float32),
            pltpu.SemaphoreType.DMA,
        ],
    )
    def k(table_hbm, gidx_hbm, out_hbm, idx_v, rows_v, sem):
        wid = lax.axis_index("s") * 2 + lax.axis_index("c")
        base = wid * _ROWS_PER_W
        pltpu.sync_copy(gidx_hbm.at[pl.ds(base, _ROWS_PER_W)], idx_v)
        for i in range(_N_IT):
            pltpu.async_copy(
                table_hbm.at[idx_v.at[pl.ds(i * _CHUNK, _CHUNK)]],
                rows_v, sem).wait()
            pltpu.sync_copy(rows_v, out_hbm.at[pl.ds(base + i * _CHUNK, _CHUNK)])

    return k(table, gidx)


_FB = 512
_NJ = _F // _FB


def _fuse_kernel(g_ref, p_ref, bt_ref, wp_ref, wb_ref, wpos_ref,
                 bp_ref, bb_ref, bpos_ref, o_ref):
    j = pl.program_id(1)
    g = g_ref[0]
    p = p_ref[0]
    bt = bt_ref[0]
    pos = (j * _FB + lax.broadcasted_iota(jnp.int32, (_FB, 1), 0)
           ).astype(jnp.float32)
    bias = bp_ref[...] + bb_ref[...] + bpos_ref[...]
    o_ref[0] = (g + p * wp_ref[...] + bt * wb_ref[...]
                + pos * wpos_ref[...] + bias)


def _fuse(g, pitch, beats, W_pitch, b_pitch, W_pos, b_pos, W_beats, b_beats):
    vec = lambda: pl.BlockSpec((1, _H), lambda b, j: (0, 0))
    return pl.pallas_call(
        _fuse_kernel,
        grid=(_B, _NJ),
        in_specs=[
            pl.BlockSpec((1, _FB, _H), lambda b, j: (b, j, 0)),
            pl.BlockSpec((1, _FB, 1), lambda b, j: (b, j, 0)),
            pl.BlockSpec((1, _FB, 1), lambda b, j: (b, j, 0)),
            vec(), vec(), vec(), vec(), vec(), vec(),
        ],
        out_specs=pl.BlockSpec((1, _FB, _H), lambda b, j: (b, j, 0)),
        out_shape=jax.ShapeDtypeStruct((_B, _F, _H), jnp.float32),
    )(g, pitch.reshape(_B, _F, 1), beats.reshape(_B, _F, 1),
      W_pitch, W_beats, W_pos,
      b_pitch.reshape(1, _H), b_beats.reshape(1, _H), b_pos.reshape(1, _H))


def kernel(encoder_out, align_phone, pitch, beats,
           W_pitch, b_pitch, W_pos, b_pos, W_beats, b_beats):
    ap = align_phone.astype(jnp.int32)
    gidx = _build_indices(ap).reshape(_B * _F)
    table = encoder_out.reshape(_B * _P, _H)
    g = _sc_gather(table, gidx).reshape(_B, _F, _H)
    return _fuse(g, pitch, beats, W_pitch, b_pitch, W_pos, b_pos,
                 W_beats, b_beats)
